# baseline (speedup 1.0000x reference)
import jax
import jax.numpy as jnp
from jax.experimental import pallas as pl

M = 100000
D = 128
_BLK = 1000


def _clear_body(mem_ref, present_ref, out_ref):
    out_ref[...] = mem_ref[...] * (1.0 - present_ref[...])


def kernel(mem, msgs, dst_ids, query_ids):
    present = jnp.zeros((M, 1), dtype=jnp.float32).at[dst_ids].set(1.0)
    cleared = pl.pallas_call(
        _clear_body,
        grid=(M // _BLK,),
        in_specs=[
            pl.BlockSpec((_BLK, D), lambda i: (i, 0)),
            pl.BlockSpec((_BLK, 1), lambda i: (i, 0)),
        ],
        out_specs=pl.BlockSpec((_BLK, D), lambda i: (i, 0)),
        out_shape=jax.ShapeDtypeStruct((M, D), jnp.float32),
    )(mem, present)
    new_mem = cleared.at[dst_ids].add(msgs)
    return jnp.take(new_mem, query_ids, axis=0)
